# Initial kernel scaffold; baseline (speedup 1.0000x reference)
import jax
import jax.numpy as jnp
from jax.experimental import pallas as pl


def kernel(inputs, embed_weight):
    raise NotImplementedError("write your pallas kernel here")


# speedup vs baseline: 1.0153x; 1.0153x over previous
import functools

import jax
import jax.numpy as jnp
from jax import lax
from jax.experimental import pallas as pl
from jax.experimental.pallas import tpu as pltpu
from jax.experimental.pallas import tpu_sc as plsc

M, N, K = 16384, 8192, 256
BM = 1024
BNP, NJ = 2736, 3
NP = BNP * NJ
NI = M // BM
BC = 1024
NC_BLK = N // BC
NUM_EMB_F = float(N)


def _rbf16(x):
    xi = lax.bitcast_convert_type(x, jnp.uint32)
    lsb = jnp.bitwise_and(jnp.right_shift(xi, 16), jnp.uint32(1))
    r = jnp.bitwise_and(xi + jnp.uint32(0x7FFF) + lsb, jnp.uint32(0xFFFF0000))
    return lax.bitcast_convert_type(r, jnp.float32)


def _a_body(x_ref, x2_ref, e_ref, e2_ref, idx_ref, gv_ref, gi_ref):
    j = pl.program_id(1)

    @pl.when(j == 0)
    def _():
        gv_ref[...] = jnp.full_like(gv_ref, jnp.inf)
        gi_ref[...] = jnp.zeros_like(gi_ref)

    a = x_ref[...]
    b = e_ref[...]
    x2 = x2_ref[...]
    e2 = e2_ref[0, 0, :]
    mm = lax.dot_general(a, b, (((1,), (1,)), ((), ())),
                         preferred_element_type=jnp.float32)
    d2 = jnp.maximum((x2 + e2[None, :]) - 2.0 * mm, 0.0)
    dist = jnp.sqrt(d2 + 1e-12)

    cols = lax.broadcasted_iota(jnp.int32, (BM, BNP), 1) + j * BNP
    bmin = jnp.min(dist, axis=1, keepdims=True)
    bidx = jnp.min(jnp.where(dist == bmin, cols, jnp.int32(2**30)),
                   axis=1, keepdims=True)
    take = bmin < gv_ref[...]
    gi_ref[...] = jnp.where(take, bidx, gi_ref[...])
    gv_ref[...] = jnp.where(take, _rbf16(bmin), gv_ref[...])

    @pl.when(j == NJ - 1)
    def _():
        idx_ref[0, :, :] = gi_ref[...]


def _phase_a(flat, x2, e_pad, e2_3d):
    idx3 = pl.pallas_call(
        _a_body,
        grid=(NI, NJ),
        in_specs=[
            pl.BlockSpec((BM, K), lambda i, j: (i, 0)),
            pl.BlockSpec((BM, 1), lambda i, j: (i, 0)),
            pl.BlockSpec((BNP, K), lambda i, j: (j, 0)),
            pl.BlockSpec((1, 8, BNP), lambda i, j: (j, 0, 0)),
        ],
        out_specs=pl.BlockSpec((1, BM, 1), lambda i, j: (i, 0, 0)),
        out_shape=jax.ShapeDtypeStruct((NI, BM, 1), jnp.int32),
        scratch_shapes=[
            pltpu.VMEM((BM, 1), jnp.float32),
            pltpu.VMEM((BM, 1), jnp.int32),
        ],
    )(flat, x2, e_pad, e2_3d)
    return idx3


def _sc_gather(table, idx):
    info = plsc.get_sparse_core_info()
    ncores, nsub = info.num_cores, info.num_subcores
    nw = ncores * nsub
    b_per_w = M // nw
    chunk = 128
    nchunk = b_per_w // chunk
    mesh = plsc.VectorSubcoreMesh(core_axis_name="c", subcore_axis_name="s")

    @functools.partial(
        pl.kernel, mesh=mesh,
        out_type=jax.ShapeDtypeStruct((M, K), jnp.float32),
        scratch_types=[
            pltpu.VMEM((b_per_w,), jnp.int32),
            pltpu.VMEM((chunk, K), jnp.float32),
            pltpu.VMEM((chunk, K), jnp.float32),
            pltpu.SemaphoreType.DMA,
            pltpu.SemaphoreType.DMA,
        ],
    )
    def k(table_hbm, idx_hbm, out_hbm, idx_v, buf0, buf1, sem0, sem1):
        wid = lax.axis_index("s") * ncores + lax.axis_index("c")
        base = wid * b_per_w
        pltpu.sync_copy(idx_hbm.at[pl.ds(base, b_per_w)], idx_v)
        bufs = (buf0, buf1)
        sems = (sem0, sem1)
        cps = []
        for c in range(nchunk):
            cp = pltpu.async_copy(
                table_hbm.at[idx_v.at[pl.ds(c * chunk, chunk)]],
                bufs[c % 2], sems[c % 2])
            cps.append(cp)
            if c >= 1:
                cps[c - 1].wait()
                pltpu.sync_copy(bufs[(c - 1) % 2],
                                out_hbm.at[pl.ds(base + (c - 1) * chunk, chunk)])
        cps[nchunk - 1].wait()
        pltpu.sync_copy(bufs[(nchunk - 1) % 2],
                        out_hbm.at[pl.ds(base + (nchunk - 1) * chunk, chunk)])

    return k(table, idx)


def _c_body(ni_ref, nj_ref, part_ref, acc_ref):
    bi = pl.program_id(0)
    bj = pl.program_id(1)

    @pl.when(jnp.logical_and(bi == 0, bj == 0))
    def _():
        acc_ref[...] = jnp.zeros_like(acc_ref)

    a = ni_ref[...]
    b = nj_ref[...]
    s = lax.dot_general(a, b, (((1,), (1,)), ((), ())),
                        preferred_element_type=jnp.float32)
    ri = lax.broadcasted_iota(jnp.int32, (BC, BC), 0) + bi * BC
    ci = lax.broadcasted_iota(jnp.int32, (BC, BC), 1) + bj * BC
    s = jnp.where(ri == ci, s - 1.0, s)
    t = jnp.clip(jnp.abs(s), 0.1, None)
    acc_ref[...] += jnp.sum(t * t, axis=1, keepdims=True)

    @pl.when(jnp.logical_and(bi == NC_BLK - 1, bj == NC_BLK - 1))
    def _():
        part_ref[...] = acc_ref[...]


def _phase_c(en):
    return pl.pallas_call(
        _c_body,
        grid=(NC_BLK, NC_BLK),
        in_specs=[
            pl.BlockSpec((BC, K), lambda i, j: (i, 0)),
            pl.BlockSpec((BC, K), lambda i, j: (j, 0)),
        ],
        out_specs=pl.BlockSpec((BC, 1), lambda i, j: (0, 0)),
        out_shape=jax.ShapeDtypeStruct((BC, 1), jnp.float32),
        scratch_shapes=[pltpu.VMEM((BC, 1), jnp.float32)],
    )(en, en)


def _n_body(e_ref, en_ref):
    b = e_ref[...]
    nrm = jnp.sqrt(jnp.sum(b * b, axis=1, keepdims=True))
    en_ref[...] = b / jnp.maximum(nrm, 1e-12)


def _phase_n(e):
    return pl.pallas_call(
        _n_body,
        grid=(NC_BLK,),
        in_specs=[pl.BlockSpec((BC, K), lambda i: (i, 0))],
        out_specs=pl.BlockSpec((BC, K), lambda i: (i, 0)),
        out_shape=jax.ShapeDtypeStruct((N, K), jnp.float32),
    )(e)


def _d_body(x_ref, q_ref, idx_ref, divp_ref, qst_ref, loss_ref,
            com_ref, hist_ref):
    i = pl.program_id(0)

    @pl.when(i == 0)
    def _():
        com_ref[...] = jnp.zeros_like(com_ref)
        hist_ref[...] = jnp.zeros_like(hist_ref)

    x = x_ref[...]
    q = q_ref[...]
    dq = q - x
    qst_ref[...] = x + dq
    com_ref[...] += jnp.sum(dq * dq, axis=1, keepdims=True)

    idx = idx_ref[0, :, :]
    for c in range(8):
        cio = lax.broadcasted_iota(jnp.int32, (BM, 1024), 1) + c * 1024
        eq = jnp.where(idx == cio, 1.0, 0.0)
        hist_ref[0:1, c * 1024:(c + 1) * 1024] += jnp.sum(
            eq, axis=0, keepdims=True)

    @pl.when(i == NI - 1)
    def _():
        counts = hist_ref[0:1, :]
        avg = counts * jnp.float32(1.0 / M)
        u = jnp.float32(1.0 / NUM_EMB_F)
        ent = jnp.log(u) - u * jnp.sum(jnp.log(avg + 1e-10))
        com = jnp.sum(com_ref[...]) * jnp.float32(1.0 / (M * K))
        div = jnp.sum(divp_ref[...]) * jnp.float32(1.0 / (NUM_EMB_F * NUM_EMB_F))
        total = 0.25 * com + 0.5 * div + 0.5 * ent
        loss_ref[...] = jnp.full_like(loss_ref, total)


def _phase_d(flat, q, idx3, divp):
    return pl.pallas_call(
        _d_body,
        grid=(NI,),
        in_specs=[
            pl.BlockSpec((BM, K), lambda i: (i, 0)),
            pl.BlockSpec((BM, K), lambda i: (i, 0)),
            pl.BlockSpec((1, BM, 1), lambda i: (i, 0, 0)),
            pl.BlockSpec((BC, 1), lambda i: (0, 0)),
        ],
        out_specs=[
            pl.BlockSpec((BM, K), lambda i: (i, 0)),
            pl.BlockSpec((8, 128), lambda i: (0, 0)),
        ],
        out_shape=[
            jax.ShapeDtypeStruct((M, K), jnp.float32),
            jax.ShapeDtypeStruct((8, 128), jnp.float32),
        ],
        scratch_shapes=[
            pltpu.VMEM((BM, 1), jnp.float32),
            pltpu.VMEM((8, N), jnp.float32),
        ],
    )(flat, q, idx3, divp)


def kernel(inputs, embed_weight):
    input_shape = inputs.shape
    flat = inputs.reshape(-1, K)
    x2 = jnp.sum(flat * flat, axis=1, keepdims=True)
    e2 = jnp.sum(embed_weight * embed_weight, axis=1)
    e_pad = jnp.concatenate(
        [embed_weight, jnp.zeros((NP - N, K), jnp.float32)], axis=0)
    e2_pad = jnp.concatenate([e2, jnp.full((NP - N,), jnp.inf, jnp.float32)])
    e2_3d = jnp.broadcast_to(e2_pad.reshape(NJ, 1, BNP), (NJ, 8, BNP))

    idx3 = _phase_a(flat, x2, e_pad, e2_3d)
    encoding_indices = idx3.reshape(M)

    quantized = _sc_gather(embed_weight, encoding_indices)

    en = _phase_n(embed_weight)
    divp = _phase_c(en)

    qst, loss_tile = _phase_d(flat, quantized, idx3, divp)
    total_loss = loss_tile[0, 0]
    return qst.reshape(input_shape), total_loss, encoding_indices


# speedup vs baseline: 1.0884x; 1.0721x over previous
import functools

import jax
import jax.numpy as jnp
from jax import lax
from jax.experimental import pallas as pl
from jax.experimental.pallas import tpu as pltpu
from jax.experimental.pallas import tpu_sc as plsc

M, N, K = 16384, 8192, 256
BM = 1024
BNP, NJ = 2736, 3
NP = BNP * NJ
NI = M // BM
BC = 1024
NC_BLK = N // BC
NUM_EMB_F = float(N)


def _rbf16(x):
    xi = lax.bitcast_convert_type(x, jnp.uint32)
    lsb = jnp.bitwise_and(jnp.right_shift(xi, 16), jnp.uint32(1))
    r = jnp.bitwise_and(xi + jnp.uint32(0x7FFF) + lsb, jnp.uint32(0xFFFF0000))
    return lax.bitcast_convert_type(r, jnp.float32)


def _a_body(x_ref, x2_ref, e_ref, e2_ref, idx_ref, gv_ref, gi_ref):
    j = pl.program_id(1)

    @pl.when(j == 0)
    def _():
        gv_ref[...] = jnp.full_like(gv_ref, jnp.inf)
        gi_ref[...] = jnp.zeros_like(gi_ref)

    a = x_ref[...]
    b = e_ref[...]
    x2 = x2_ref[...]
    e2 = e2_ref[0, 0, :]
    mm = lax.dot_general(a, b, (((1,), (1,)), ((), ())),
                         preferred_element_type=jnp.float32)
    d2 = jnp.maximum((x2 + e2[None, :]) - 2.0 * mm, 0.0)
    dist = jnp.sqrt(d2 + 1e-12)

    cols = lax.broadcasted_iota(jnp.int32, (BM, BNP), 1) + j * BNP
    bmin = jnp.min(dist, axis=1, keepdims=True)
    bidx = jnp.min(jnp.where(dist == bmin, cols, jnp.int32(2**30)),
                   axis=1, keepdims=True)
    take = bmin < gv_ref[...]
    gi_ref[...] = jnp.where(take, bidx, gi_ref[...])
    gv_ref[...] = jnp.where(take, _rbf16(bmin), gv_ref[...])

    @pl.when(j == NJ - 1)
    def _():
        idx_ref[0, :, :] = gi_ref[...]


def _phase_a(flat, x2, e_pad, e2_3d):
    idx3 = pl.pallas_call(
        _a_body,
        grid=(NI, NJ),
        in_specs=[
            pl.BlockSpec((BM, K), lambda i, j: (i, 0)),
            pl.BlockSpec((BM, 1), lambda i, j: (i, 0)),
            pl.BlockSpec((BNP, K), lambda i, j: (j, 0)),
            pl.BlockSpec((1, 8, BNP), lambda i, j: (j, 0, 0)),
        ],
        out_specs=pl.BlockSpec((1, BM, 1), lambda i, j: (i, 0, 0)),
        out_shape=jax.ShapeDtypeStruct((NI, BM, 1), jnp.int32),
        scratch_shapes=[
            pltpu.VMEM((BM, 1), jnp.float32),
            pltpu.VMEM((BM, 1), jnp.int32),
        ],
    )(flat, x2, e_pad, e2_3d)
    return idx3


def _sc_gather(table, idx):
    info = plsc.get_sparse_core_info()
    ncores, nsub = info.num_cores, info.num_subcores
    nw = ncores * nsub
    b_per_w = M // nw
    chunk = 128
    nchunk = b_per_w // chunk
    mesh = plsc.VectorSubcoreMesh(core_axis_name="c", subcore_axis_name="s")

    @functools.partial(
        pl.kernel, mesh=mesh,
        out_type=jax.ShapeDtypeStruct((M, K), jnp.float32),
        scratch_types=[
            pltpu.VMEM((b_per_w,), jnp.int32),
            pltpu.VMEM((chunk, K), jnp.float32),
            pltpu.VMEM((chunk, K), jnp.float32),
            pltpu.SemaphoreType.DMA,
            pltpu.SemaphoreType.DMA,
        ],
    )
    def k(table_hbm, idx_hbm, out_hbm, idx_v, buf0, buf1, sem0, sem1):
        wid = lax.axis_index("s") * ncores + lax.axis_index("c")
        base = wid * b_per_w
        pltpu.sync_copy(idx_hbm.at[pl.ds(base, b_per_w)], idx_v)
        bufs = (buf0, buf1)
        sems = (sem0, sem1)
        cps = []
        for c in range(nchunk):
            cp = pltpu.async_copy(
                table_hbm.at[idx_v.at[pl.ds(c * chunk, chunk)]],
                bufs[c % 2], sems[c % 2])
            cps.append(cp)
            if c >= 1:
                cps[c - 1].wait()
                pltpu.sync_copy(bufs[(c - 1) % 2],
                                out_hbm.at[pl.ds(base + (c - 1) * chunk, chunk)])
        cps[nchunk - 1].wait()
        pltpu.sync_copy(bufs[(nchunk - 1) % 2],
                        out_hbm.at[pl.ds(base + (nchunk - 1) * chunk, chunk)])

    return k(table, idx)


_PAIRS = [(i, j) for i in range(NC_BLK) for j in range(i, NC_BLK)]
_NPAIR = len(_PAIRS)


def _c_body(bi_ref, bj_ref, ni_ref, nj_ref, part_ref, acc_ref):
    p = pl.program_id(0)

    @pl.when(p == 0)
    def _():
        acc_ref[...] = jnp.zeros_like(acc_ref)

    bi = bi_ref[p]
    bj = bj_ref[p]
    a = ni_ref[...]
    b = nj_ref[...]
    s = lax.dot_general(a, b, (((1,), (1,)), ((), ())),
                        preferred_element_type=jnp.float32)
    ri = lax.broadcasted_iota(jnp.int32, (BC, BC), 0) + bi * BC
    ci = lax.broadcasted_iota(jnp.int32, (BC, BC), 1) + bj * BC
    s = jnp.where(ri == ci, s - 1.0, s)
    t = jnp.clip(jnp.abs(s), 0.1, None)
    w = jnp.where(bi == bj, 1.0, 2.0).astype(jnp.float32)
    acc_ref[...] += w * jnp.sum(t * t, axis=1, keepdims=True)

    @pl.when(p == _NPAIR - 1)
    def _():
        part_ref[...] = acc_ref[...]


def _phase_c(en):
    bi_arr = jnp.asarray([p[0] for p in _PAIRS], jnp.int32)
    bj_arr = jnp.asarray([p[1] for p in _PAIRS], jnp.int32)
    grid_spec = pltpu.PrefetchScalarGridSpec(
        num_scalar_prefetch=2,
        grid=(_NPAIR,),
        in_specs=[
            pl.BlockSpec((BC, K), lambda p, bi, bj: (bi[p], 0)),
            pl.BlockSpec((BC, K), lambda p, bi, bj: (bj[p], 0)),
        ],
        out_specs=pl.BlockSpec((BC, 1), lambda p, bi, bj: (0, 0)),
        scratch_shapes=[pltpu.VMEM((BC, 1), jnp.float32)],
    )
    return pl.pallas_call(
        _c_body,
        grid_spec=grid_spec,
        out_shape=jax.ShapeDtypeStruct((BC, 1), jnp.float32),
    )(bi_arr, bj_arr, en, en)


def _n_body(e_ref, en_ref):
    b = e_ref[...]
    nrm = jnp.sqrt(jnp.sum(b * b, axis=1, keepdims=True))
    en_ref[...] = b / jnp.maximum(nrm, 1e-12)


def _phase_n(e):
    return pl.pallas_call(
        _n_body,
        grid=(NC_BLK,),
        in_specs=[pl.BlockSpec((BC, K), lambda i: (i, 0))],
        out_specs=pl.BlockSpec((BC, K), lambda i: (i, 0)),
        out_shape=jax.ShapeDtypeStruct((N, K), jnp.float32),
    )(e)


def _d_body(x_ref, q_ref, idx_ref, divp_ref, qst_ref, loss_ref,
            com_ref, hist_ref):
    i = pl.program_id(0)

    @pl.when(i == 0)
    def _():
        com_ref[...] = jnp.zeros_like(com_ref)
        hist_ref[...] = jnp.zeros_like(hist_ref)

    x = x_ref[...]
    q = q_ref[...]
    dq = q - x
    qst_ref[...] = x + dq
    com_ref[...] += jnp.sum(dq * dq, axis=1, keepdims=True)

    idx = idx_ref[0, :, :]
    for c in range(8):
        cio = lax.broadcasted_iota(jnp.int32, (BM, 1024), 1) + c * 1024
        eq = jnp.where(idx == cio, 1.0, 0.0)
        hist_ref[0:1, c * 1024:(c + 1) * 1024] += jnp.sum(
            eq, axis=0, keepdims=True)

    @pl.when(i == NI - 1)
    def _():
        counts = hist_ref[0:1, :]
        avg = counts * jnp.float32(1.0 / M)
        u = jnp.float32(1.0 / NUM_EMB_F)
        ent = jnp.log(u) - u * jnp.sum(jnp.log(avg + 1e-10))
        com = jnp.sum(com_ref[...]) * jnp.float32(1.0 / (M * K))
        div = jnp.sum(divp_ref[...]) * jnp.float32(1.0 / (NUM_EMB_F * NUM_EMB_F))
        total = 0.25 * com + 0.5 * div + 0.5 * ent
        loss_ref[...] = jnp.full_like(loss_ref, total)


def _phase_d(flat, q, idx3, divp):
    return pl.pallas_call(
        _d_body,
        grid=(NI,),
        in_specs=[
            pl.BlockSpec((BM, K), lambda i: (i, 0)),
            pl.BlockSpec((BM, K), lambda i: (i, 0)),
            pl.BlockSpec((1, BM, 1), lambda i: (i, 0, 0)),
            pl.BlockSpec((BC, 1), lambda i: (0, 0)),
        ],
        out_specs=[
            pl.BlockSpec((BM, K), lambda i: (i, 0)),
            pl.BlockSpec((8, 128), lambda i: (0, 0)),
        ],
        out_shape=[
            jax.ShapeDtypeStruct((M, K), jnp.float32),
            jax.ShapeDtypeStruct((8, 128), jnp.float32),
        ],
        scratch_shapes=[
            pltpu.VMEM((BM, 1), jnp.float32),
            pltpu.VMEM((8, N), jnp.float32),
        ],
    )(flat, q, idx3, divp)


def kernel(inputs, embed_weight):
    input_shape = inputs.shape
    flat = inputs.reshape(-1, K)
    x2 = jnp.sum(flat * flat, axis=1, keepdims=True)
    e2 = jnp.sum(embed_weight * embed_weight, axis=1)
    e_pad = jnp.concatenate(
        [embed_weight, jnp.zeros((NP - N, K), jnp.float32)], axis=0)
    e2_pad = jnp.concatenate([e2, jnp.full((NP - N,), jnp.inf, jnp.float32)])
    e2_3d = jnp.broadcast_to(e2_pad.reshape(NJ, 1, BNP), (NJ, 8, BNP))

    idx3 = _phase_a(flat, x2, e_pad, e2_3d)
    encoding_indices = idx3.reshape(M)

    quantized = _sc_gather(embed_weight, encoding_indices)

    en = _phase_n(embed_weight)
    divp = _phase_c(en)

    qst, loss_tile = _phase_d(flat, quantized, idx3, divp)
    total_loss = loss_tile[0, 0]
    return qst.reshape(input_shape), total_loss, encoding_indices


# speedup vs baseline: 1.1371x; 1.0447x over previous
import functools

import jax
import jax.numpy as jnp
from jax import lax
from jax.experimental import pallas as pl
from jax.experimental.pallas import tpu as pltpu
from jax.experimental.pallas import tpu_sc as plsc

M, N, K = 16384, 8192, 256
BM = 2048
BNP, NJ = 2736, 3
NP = BNP * NJ
NI = M // BM
BC = 1024
NC_BLK = N // BC
NUM_EMB_F = float(N)


def _rbf16(x):
    xi = lax.bitcast_convert_type(x, jnp.uint32)
    lsb = jnp.bitwise_and(jnp.right_shift(xi, 16), jnp.uint32(1))
    r = jnp.bitwise_and(xi + jnp.uint32(0x7FFF) + lsb, jnp.uint32(0xFFFF0000))
    return lax.bitcast_convert_type(r, jnp.float32)


def _a_body(x_ref, x2_ref, e_ref, e2_ref, idx_ref, gv_ref, gi_ref):
    j = pl.program_id(1)

    @pl.when(j == 0)
    def _():
        gv_ref[...] = jnp.full_like(gv_ref, jnp.inf)
        gi_ref[...] = jnp.zeros_like(gi_ref)

    a = x_ref[...]
    b = e_ref[...]
    x2 = x2_ref[...]
    e2 = e2_ref[0, 0, :]
    mm = lax.dot_general(a, b, (((1,), (1,)), ((), ())),
                         preferred_element_type=jnp.float32)
    d2 = jnp.maximum((x2 + e2[None, :]) - 2.0 * mm, 0.0)
    dist = jnp.sqrt(d2 + 1e-12)

    cols = lax.broadcasted_iota(jnp.int32, (BM, BNP), 1) + j * BNP
    bmin = jnp.min(dist, axis=1, keepdims=True)
    bidx = jnp.min(jnp.where(dist == bmin, cols, jnp.int32(2**30)),
                   axis=1, keepdims=True)
    take = bmin < gv_ref[...]
    gi_ref[...] = jnp.where(take, bidx, gi_ref[...])
    gv_ref[...] = jnp.where(take, _rbf16(bmin), gv_ref[...])

    @pl.when(j == NJ - 1)
    def _():
        idx_ref[0, :, :] = gi_ref[...]


def _phase_a(flat, x2, e_pad, e2_3d):
    idx3 = pl.pallas_call(
        _a_body,
        grid=(NI, NJ),
        in_specs=[
            pl.BlockSpec((BM, K), lambda i, j: (i, 0)),
            pl.BlockSpec((BM, 1), lambda i, j: (i, 0)),
            pl.BlockSpec((BNP, K), lambda i, j: (j, 0)),
            pl.BlockSpec((1, 8, BNP), lambda i, j: (j, 0, 0)),
        ],
        out_specs=pl.BlockSpec((1, BM, 1), lambda i, j: (i, 0, 0)),
        out_shape=jax.ShapeDtypeStruct((NI, BM, 1), jnp.int32),
        scratch_shapes=[
            pltpu.VMEM((BM, 1), jnp.float32),
            pltpu.VMEM((BM, 1), jnp.int32),
        ],
    )(flat, x2, e_pad, e2_3d)
    return idx3


def _sc_gather(table, idx):
    info = plsc.get_sparse_core_info()
    ncores, nsub = info.num_cores, info.num_subcores
    nw = ncores * nsub
    b_per_w = M // nw
    chunk = 128
    nchunk = b_per_w // chunk
    mesh = plsc.VectorSubcoreMesh(core_axis_name="c", subcore_axis_name="s")

    @functools.partial(
        pl.kernel, mesh=mesh,
        out_type=jax.ShapeDtypeStruct((M, K), jnp.float32),
        scratch_types=[
            pltpu.VMEM((b_per_w,), jnp.int32),
            pltpu.VMEM((chunk, K), jnp.float32),
            pltpu.VMEM((chunk, K), jnp.float32),
            pltpu.SemaphoreType.DMA,
            pltpu.SemaphoreType.DMA,
        ],
    )
    def k(table_hbm, idx_hbm, out_hbm, idx_v, buf0, buf1, sem0, sem1):
        wid = lax.axis_index("s") * ncores + lax.axis_index("c")
        base = wid * b_per_w
        pltpu.sync_copy(idx_hbm.at[pl.ds(base, b_per_w)], idx_v)
        bufs = (buf0, buf1)
        sems = (sem0, sem1)
        cps = []
        for c in range(nchunk):
            cp = pltpu.async_copy(
                table_hbm.at[idx_v.at[pl.ds(c * chunk, chunk)]],
                bufs[c % 2], sems[c % 2])
            cps.append(cp)
            if c >= 1:
                cps[c - 1].wait()
                pltpu.sync_copy(bufs[(c - 1) % 2],
                                out_hbm.at[pl.ds(base + (c - 1) * chunk, chunk)])
        cps[nchunk - 1].wait()
        pltpu.sync_copy(bufs[(nchunk - 1) % 2],
                        out_hbm.at[pl.ds(base + (nchunk - 1) * chunk, chunk)])

    return k(table, idx)


_PAIRS = [(i, j) for i in range(NC_BLK) for j in range(i, NC_BLK)]
_NPAIR = len(_PAIRS)


def _c_body(bi_ref, bj_ref, ni_ref, nj_ref, part_ref, acc_ref):
    p = pl.program_id(0)

    @pl.when(p == 0)
    def _():
        acc_ref[...] = jnp.zeros_like(acc_ref)

    bi = bi_ref[p]
    bj = bj_ref[p]
    a = ni_ref[...]
    b = nj_ref[...]
    s = lax.dot_general(a, b, (((1,), (1,)), ((), ())),
                        preferred_element_type=jnp.float32)
    ri = lax.broadcasted_iota(jnp.int32, (BC, BC), 0) + bi * BC
    ci = lax.broadcasted_iota(jnp.int32, (BC, BC), 1) + bj * BC
    s = jnp.where(ri == ci, s - 1.0, s)
    t = jnp.clip(jnp.abs(s), 0.1, None)
    w = jnp.where(bi == bj, 1.0, 2.0).astype(jnp.float32)
    acc_ref[...] += w * jnp.sum(t * t, axis=1, keepdims=True)

    @pl.when(p == _NPAIR - 1)
    def _():
        part_ref[...] = acc_ref[...]


def _phase_c(en):
    bi_arr = jnp.asarray([p[0] for p in _PAIRS], jnp.int32)
    bj_arr = jnp.asarray([p[1] for p in _PAIRS], jnp.int32)
    grid_spec = pltpu.PrefetchScalarGridSpec(
        num_scalar_prefetch=2,
        grid=(_NPAIR,),
        in_specs=[
            pl.BlockSpec((BC, K), lambda p, bi, bj: (bi[p], 0)),
            pl.BlockSpec((BC, K), lambda p, bi, bj: (bj[p], 0)),
        ],
        out_specs=pl.BlockSpec((BC, 1), lambda p, bi, bj: (0, 0)),
        scratch_shapes=[pltpu.VMEM((BC, 1), jnp.float32)],
    )
    return pl.pallas_call(
        _c_body,
        grid_spec=grid_spec,
        out_shape=jax.ShapeDtypeStruct((BC, 1), jnp.float32),
    )(bi_arr, bj_arr, en, en)


def _n_body(e_ref, en_ref):
    b = e_ref[...]
    nrm = jnp.sqrt(jnp.sum(b * b, axis=1, keepdims=True))
    en_ref[...] = b / jnp.maximum(nrm, 1e-12)


def _phase_n(e):
    return pl.pallas_call(
        _n_body,
        grid=(NC_BLK,),
        in_specs=[pl.BlockSpec((BC, K), lambda i: (i, 0))],
        out_specs=pl.BlockSpec((BC, K), lambda i: (i, 0)),
        out_shape=jax.ShapeDtypeStruct((N, K), jnp.float32),
    )(e)


def _d_body(x_ref, q_ref, idx_ref, divp_ref, qst_ref, loss_ref,
            com_ref, hist_ref):
    i = pl.program_id(0)

    @pl.when(i == 0)
    def _():
        com_ref[...] = jnp.zeros_like(com_ref)
        hist_ref[...] = jnp.zeros_like(hist_ref)

    x = x_ref[...]
    q = q_ref[...]
    dq = q - x
    qst_ref[...] = x + dq
    com_ref[...] += jnp.sum(dq * dq, axis=1, keepdims=True)

    idx = idx_ref[0, :, :]
    for c in range(8):
        cio = lax.broadcasted_iota(jnp.int32, (BM, 1024), 1) + c * 1024
        eq = jnp.where(idx == cio, 1.0, 0.0)
        hist_ref[0:1, c * 1024:(c + 1) * 1024] += jnp.sum(
            eq, axis=0, keepdims=True)

    @pl.when(i == NI - 1)
    def _():
        counts = hist_ref[0:1, :]
        avg = counts * jnp.float32(1.0 / M)
        u = jnp.float32(1.0 / NUM_EMB_F)
        ent = jnp.log(u) - u * jnp.sum(jnp.log(avg + 1e-10))
        com = jnp.sum(com_ref[...]) * jnp.float32(1.0 / (M * K))
        div = jnp.sum(divp_ref[...]) * jnp.float32(1.0 / (NUM_EMB_F * NUM_EMB_F))
        total = 0.25 * com + 0.5 * div + 0.5 * ent
        loss_ref[...] = jnp.full_like(loss_ref, total)


def _phase_d(flat, q, idx3, divp):
    return pl.pallas_call(
        _d_body,
        grid=(NI,),
        in_specs=[
            pl.BlockSpec((BM, K), lambda i: (i, 0)),
            pl.BlockSpec((BM, K), lambda i: (i, 0)),
            pl.BlockSpec((1, BM, 1), lambda i: (i, 0, 0)),
            pl.BlockSpec((BC, 1), lambda i: (0, 0)),
        ],
        out_specs=[
            pl.BlockSpec((BM, K), lambda i: (i, 0)),
            pl.BlockSpec((8, 128), lambda i: (0, 0)),
        ],
        out_shape=[
            jax.ShapeDtypeStruct((M, K), jnp.float32),
            jax.ShapeDtypeStruct((8, 128), jnp.float32),
        ],
        scratch_shapes=[
            pltpu.VMEM((BM, 1), jnp.float32),
            pltpu.VMEM((8, N), jnp.float32),
        ],
    )(flat, q, idx3, divp)


def kernel(inputs, embed_weight):
    input_shape = inputs.shape
    flat = inputs.reshape(-1, K)
    x2 = jnp.sum(flat * flat, axis=1, keepdims=True)
    e2 = jnp.sum(embed_weight * embed_weight, axis=1)
    e_pad = jnp.concatenate(
        [embed_weight, jnp.zeros((NP - N, K), jnp.float32)], axis=0)
    e2_pad = jnp.concatenate([e2, jnp.full((NP - N,), jnp.inf, jnp.float32)])
    e2_3d = jnp.broadcast_to(e2_pad.reshape(NJ, 1, BNP), (NJ, 8, BNP))

    idx3 = _phase_a(flat, x2, e_pad, e2_3d)
    encoding_indices = idx3.reshape(M)

    quantized = _sc_gather(embed_weight, encoding_indices)

    en = _phase_n(embed_weight)
    divp = _phase_c(en)

    qst, loss_tile = _phase_d(flat, quantized, idx3, divp)
    total_loss = loss_tile[0, 0]
    return qst.reshape(input_shape), total_loss, encoding_indices
